# baseline (speedup 1.0000x reference)
import functools

import jax
import jax.numpy as jnp
from jax import lax
from jax.experimental import pallas as pl
from jax.experimental.pallas import tpu as pltpu
from jax.experimental.pallas import tpu_sc as plsc

N = 10000
E = 320000
D = 128
NC, NS, L = 2, 16, 16
NW = NC * NS
N_PAD = 10240
HI = N_PAD // 128
RPT = N_PAD // NS
B = 64
NB = 160
E_PAD = NW * B * NB
_CH = 64
_NCH = RPT // _CH
ECB = 8
ECE = ECB * B
NCHK = NB // ECB


@functools.lru_cache(maxsize=None)
def _make_sc_agg():
    mesh = plsc.VectorSubcoreMesh(core_axis_name="c", subcore_axis_name="s",
                                  num_cores=NC, num_subcores=NS)
    out_type = jax.ShapeDtypeStruct((NC, N_PAD, D), jnp.float32)
    scratch = [
        pltpu.VMEM_SHARED((N_PAD, D), jnp.float32),
        [pltpu.VMEM((2 * ECB, B), jnp.int32)] * 2,
        pltpu.VMEM((ECE,), jnp.float32),
        [pltpu.VMEM((B, D), jnp.float32)] * 3,
        [pltpu.SemaphoreType.DMA] * 3,
        [pltpu.SemaphoreType.DMA] * 3,
    ]

    @functools.partial(pl.kernel, out_type=out_type, mesh=mesh,
                       scratch_types=scratch)
    def sc_agg(feat, sd2, wr, zrows, agg_out,
               aggs, sdv2, wv, rows3, semg3, sems3):
        c = lax.axis_index("c")
        s = lax.axis_index("s")
        wid = c * NS + s
        row0 = s * RPT

        pltpu.sync_copy(zrows, rows3[0])
        for j in range(_NCH):
            pltpu.async_copy(rows3[0], aggs.at[pl.ds(row0 + j * _CH, _CH)],
                             semg3[0])
        for j in range(_NCH):
            pltpu.make_async_copy(rows3[0],
                                  aggs.at[pl.ds(row0 + j * _CH, _CH)],
                                  semg3[0]).wait()
        plsc.subcore_barrier()

        def scale(rows, j):
            def g_body(g, _):
                w16 = wv[pl.ds(j * B + g * L, L)]
                for k in range(L):
                    wb = jnp.broadcast_to(w16[k], (L,))
                    e = g * L + k
                    for cc in range(D // L):
                        sl = pl.ds(cc * L, L)
                        rows[e, sl] = rows[e, sl] * wb
                return 0
            lax.fori_loop(0, B // L, g_body, 0)

        def wait_scat(b, sdv):
            pltpu.make_async_copy(rows3[b], aggs.at[sdv.at[ECB]],
                                  sems3[b]).wait()

        def chunk_work(ck, sdv, psdv, first):
            base = (wid * NCHK + ck) * 2 * ECB
            pltpu.sync_copy(sd2.at[pl.ds(base, 2 * ECB)], sdv)
            pltpu.sync_copy(wr.at[pl.ds((wid * NB + ck * ECB) * B, ECE)], wv)
            if not first:
                wait_scat(0, psdv)
            pltpu.async_copy(feat.at[sdv.at[0]], rows3[0], semg3[0])
            for j in range(ECB):
                b = j % 3
                if j + 1 < ECB:
                    nb_ = (j + 1) % 3
                    if j < 2:
                        if not first:
                            wait_scat(nb_, psdv)
                    else:
                        wait_scat(nb_, sdv)
                    pltpu.async_copy(feat.at[sdv.at[j + 1]], rows3[nb_],
                                     semg3[nb_])
                pltpu.make_async_copy(feat.at[sdv.at[j]], rows3[b],
                                      semg3[b]).wait()
                scale(rows3[b], j)
                pltpu.async_copy(rows3[b], aggs.at[sdv.at[ECB + j]], sems3[b],
                                 add=True)

        chunk_work(0, sdv2[0], sdv2[1], True)
        chunk_work(1, sdv2[1], sdv2[0], False)

        def pair_body(p, _):
            chunk_work(2 + 2 * p, sdv2[0], sdv2[1], False)
            chunk_work(3 + 2 * p, sdv2[1], sdv2[0], False)
            return 0

        lax.fori_loop(0, (NCHK - 2) // 2, pair_body, 0)

        for b in range(3):
            wait_scat(b, sdv2[1])
        plsc.subcore_barrier()

        for j in range(_NCH):
            b = j % 3
            if j >= 3:
                pltpu.make_async_copy(
                    rows3[b], agg_out.at[c, pl.ds(row0 + (j - 3) * _CH, _CH)],
                    sems3[b]).wait()
            pltpu.sync_copy(aggs.at[pl.ds(row0 + j * _CH, _CH)], rows3[b])
            pltpu.async_copy(rows3[b],
                             agg_out.at[c, pl.ds(row0 + j * _CH, _CH)],
                             sems3[b])
        for j in range(_NCH - 3, _NCH):
            pltpu.make_async_copy(rows3[j % 3],
                                  agg_out.at[c, pl.ds(row0 + j * _CH, _CH)],
                                  sems3[j % 3]).wait()

    return sc_agg


_ER = 512
_EBR = 8


def _deg_body(dst_ref, deg_ref):
    @pl.when(pl.program_id(0) == 0)
    def _():
        deg_ref[...] = jnp.zeros_like(deg_ref)
    acc = deg_ref[...]
    lanes = lax.broadcasted_iota(jnp.int32, (1, 128), 1)
    his = lax.broadcasted_iota(jnp.int32, (HI, 1), 0)
    for r in range(_EBR):
        d = dst_ref[r, :]
        oh_lo = (d[:, None] % 128 == lanes).astype(jnp.bfloat16)
        oh_hi = (d[None, :] // 128 == his).astype(jnp.bfloat16)
        acc += jnp.dot(oh_hi, oh_lo, preferred_element_type=jnp.float32)
    deg_ref[...] = acc


def _tc_degree(dst2d):
    return pl.pallas_call(
        _deg_body,
        grid=(E_PAD // (_ER * _EBR),),
        in_specs=[pl.BlockSpec((_EBR, _ER), lambda i: (i, 0))],
        out_specs=pl.BlockSpec((HI, 128), lambda i: (0, 0)),
        out_shape=jax.ShapeDtypeStruct((HI, 128), jnp.float32),
    )(dst2d)


_BR = 1024


def _tc_body(relu, agg_ref, deg_ref, w_ref, b_ref, out_ref):
    a = agg_ref[0] + agg_ref[1]
    h = jnp.dot(a, w_ref[...], preferred_element_type=jnp.float32)
    norm = 1.0 / jnp.clip(deg_ref[...], 1.0, None)
    h = h * norm + b_ref[...]
    if relu:
        h = jnp.maximum(h, 0.0)
    out_ref[...] = h


def _tc_layer(agg, deg, w, b, relu, n_out):
    return pl.pallas_call(
        functools.partial(_tc_body, relu),
        grid=(N_PAD // _BR,),
        in_specs=[
            pl.BlockSpec((NC, _BR, D), lambda i: (0, i, 0)),
            pl.BlockSpec((_BR, 1), lambda i: (i, 0)),
            pl.BlockSpec((D, D), lambda i: (0, 0)),
            pl.BlockSpec((1, D), lambda i: (0, 0)),
        ],
        out_specs=pl.BlockSpec((_BR, D), lambda i: (i, 0)),
        out_shape=jax.ShapeDtypeStruct((n_out, D), jnp.float32),
    )(agg, deg, w, b)


def kernel(x, edge_index, edge_weight, W1, b1, W2, b2):
    src = edge_index[0]
    dst = edge_index[1]
    pad = E_PAD - E
    ar = jnp.arange(pad, dtype=jnp.int32)
    src_p = jnp.concatenate([src, ar % N])
    dst_p = jnp.concatenate([dst, N + ar % (N_PAD - N)])
    w_p = jnp.concatenate([edge_weight, jnp.zeros((pad,), jnp.float32)])
    dst2d = dst_p.reshape(E_PAD // _ER, _ER)
    sd2 = jnp.concatenate(
        [src_p.reshape(NW, NCHK, ECB, B), dst_p.reshape(NW, NCHK, ECB, B)],
        axis=2).reshape(NW * NCHK * 2 * ECB, B)
    zrows = jnp.zeros((_CH, D), jnp.float32)

    agg1 = _make_sc_agg()(x, sd2, w_p, zrows)
    deg_col = _tc_degree(dst2d).reshape(N_PAD, 1)
    h = _tc_layer(agg1, deg_col, W1, b1.reshape(1, D), relu=True, n_out=N_PAD)
    agg2 = _make_sc_agg()(h, sd2, w_p, zrows)
    out = _tc_layer(agg2, deg_col, W2, b2.reshape(1, D), relu=False, n_out=N)
    return out


# speedup vs baseline: 1.1267x; 1.1267x over previous
import functools

import jax
import jax.numpy as jnp
from jax import lax
from jax.experimental import pallas as pl
from jax.experimental.pallas import tpu as pltpu
from jax.experimental.pallas import tpu_sc as plsc

N = 10000
E = 320000
D = 128
NC, NS, L = 2, 16, 16
NW = NC * NS
N_PAD = 10240
HI = N_PAD // 128
RPT = N_PAD // NS
B = 64
NB = 160
E_PAD = NW * B * NB
_CH = 64
_NCH = RPT // _CH
ECB = 8
ECE = ECB * B
NCHK = NB // ECB


@functools.lru_cache(maxsize=None)
def _make_sc_agg():
    mesh = plsc.VectorSubcoreMesh(core_axis_name="c", subcore_axis_name="s",
                                  num_cores=NC, num_subcores=NS)
    out_type = jax.ShapeDtypeStruct((NC, N_PAD, D), jnp.float32)
    REC = 3 * ECB
    scratch = [
        pltpu.VMEM_SHARED((N_PAD, D), jnp.float32),
        [pltpu.VMEM((REC, B), jnp.int32)] * 2,
        [pltpu.VMEM((B, D), jnp.float32)] * 3,
        [pltpu.SemaphoreType.DMA] * 3,
        [pltpu.SemaphoreType.DMA] * 3,
        pltpu.SemaphoreType.DMA,
    ]

    @functools.partial(pl.kernel, out_type=out_type, mesh=mesh,
                       scratch_types=scratch)
    def sc_agg(feat, sd2, zrows, agg_out,
               aggs, sdv2, rows3, semg3, sems3, sem_e):
        c = lax.axis_index("c")
        s = lax.axis_index("s")
        wid = c * NS + s
        row0 = s * RPT

        pltpu.sync_copy(zrows, rows3[0])
        for j in range(_NCH):
            pltpu.async_copy(rows3[0], aggs.at[pl.ds(row0 + j * _CH, _CH)],
                             semg3[0])
        for j in range(_NCH):
            pltpu.make_async_copy(rows3[0],
                                  aggs.at[pl.ds(row0 + j * _CH, _CH)],
                                  semg3[0]).wait()
        plsc.subcore_barrier()

        def scale(rows, sdv, j):
            def g_body(g, _):
                w16 = lax.bitcast_convert_type(
                    sdv[2 * ECB + j, pl.ds(g * L, L)], jnp.float32)
                for k in range(L):
                    wb = jnp.broadcast_to(w16[k], (L,))
                    e = g * L + k
                    for cc in range(D // L):
                        sl = pl.ds(cc * L, L)
                        rows[e, sl] = rows[e, sl] * wb
                return 0
            lax.fori_loop(0, B // L, g_body, 0)

        def wait_scat(b, sdv):
            pltpu.make_async_copy(rows3[b], aggs.at[sdv.at[ECB]],
                                  sems3[b]).wait()

        def chunk_work(ck, sdv, osdv, first):
            base = (wid * NCHK + ck) * REC
            if first:
                pltpu.sync_copy(sd2.at[pl.ds(base, REC)], sdv)
            else:
                pltpu.make_async_copy(sd2.at[pl.ds(base, REC)], sdv,
                                      sem_e).wait()
                wait_scat(0, osdv)
            pltpu.async_copy(feat.at[sdv.at[0]], rows3[0], semg3[0])
            for j in range(ECB):
                b = j % 3
                if j + 1 < ECB:
                    nb_ = (j + 1) % 3
                    if j < 2:
                        if not first:
                            wait_scat(nb_, osdv)
                    else:
                        wait_scat(nb_, sdv)
                    pltpu.async_copy(feat.at[sdv.at[j + 1]], rows3[nb_],
                                     semg3[nb_])
                if j == 2:
                    pltpu.async_copy(sd2.at[pl.ds(base + REC, REC)], osdv,
                                     sem_e)
                pltpu.make_async_copy(feat.at[sdv.at[j]], rows3[b],
                                      semg3[b]).wait()
                scale(rows3[b], sdv, j)
                pltpu.async_copy(rows3[b], aggs.at[sdv.at[ECB + j]], sems3[b],
                                 add=True)

        chunk_work(0, sdv2[0], sdv2[1], True)
        chunk_work(1, sdv2[1], sdv2[0], False)

        def pair_body(p, _):
            chunk_work(2 + 2 * p, sdv2[0], sdv2[1], False)
            chunk_work(3 + 2 * p, sdv2[1], sdv2[0], False)
            return 0

        lax.fori_loop(0, (NCHK - 2) // 2, pair_body, 0)

        for b in range(3):
            wait_scat(b, sdv2[1])
        pltpu.make_async_copy(sd2.at[pl.ds(0, REC)], sdv2[0], sem_e).wait()
        plsc.subcore_barrier()

        for j in range(_NCH):
            b = j % 3
            if j >= 3:
                pltpu.make_async_copy(
                    rows3[b], agg_out.at[c, pl.ds(row0 + (j - 3) * _CH, _CH)],
                    sems3[b]).wait()
            pltpu.sync_copy(aggs.at[pl.ds(row0 + j * _CH, _CH)], rows3[b])
            pltpu.async_copy(rows3[b],
                             agg_out.at[c, pl.ds(row0 + j * _CH, _CH)],
                             sems3[b])
        for j in range(_NCH - 3, _NCH):
            pltpu.make_async_copy(rows3[j % 3],
                                  agg_out.at[c, pl.ds(row0 + j * _CH, _CH)],
                                  sems3[j % 3]).wait()

    return sc_agg


_ER = 512
_EBR = 8


def _deg_body(dst_ref, deg_ref):
    @pl.when(pl.program_id(0) == 0)
    def _():
        deg_ref[...] = jnp.zeros_like(deg_ref)
    acc = deg_ref[...]
    lanes = lax.broadcasted_iota(jnp.int32, (1, 128), 1)
    his = lax.broadcasted_iota(jnp.int32, (HI, 1), 0)
    for r in range(_EBR):
        d = dst_ref[r, :]
        oh_lo = (d[:, None] % 128 == lanes).astype(jnp.bfloat16)
        oh_hi = (d[None, :] // 128 == his).astype(jnp.bfloat16)
        acc += jnp.dot(oh_hi, oh_lo, preferred_element_type=jnp.float32)
    deg_ref[...] = acc


def _tc_degree(dst2d):
    return pl.pallas_call(
        _deg_body,
        grid=(E_PAD // (_ER * _EBR),),
        in_specs=[pl.BlockSpec((_EBR, _ER), lambda i: (i, 0))],
        out_specs=pl.BlockSpec((HI, 128), lambda i: (0, 0)),
        out_shape=jax.ShapeDtypeStruct((HI, 128), jnp.float32),
    )(dst2d)


_BR = 1024


def _tc_body(relu, agg_ref, deg_ref, w_ref, b_ref, out_ref):
    a = agg_ref[0] + agg_ref[1]
    h = jnp.dot(a, w_ref[...], preferred_element_type=jnp.float32)
    norm = 1.0 / jnp.clip(deg_ref[...], 1.0, None)
    h = h * norm + b_ref[...]
    if relu:
        h = jnp.maximum(h, 0.0)
    out_ref[...] = h


def _tc_layer(agg, deg, w, b, relu, n_out):
    return pl.pallas_call(
        functools.partial(_tc_body, relu),
        grid=(N_PAD // _BR,),
        in_specs=[
            pl.BlockSpec((NC, _BR, D), lambda i: (0, i, 0)),
            pl.BlockSpec((_BR, 1), lambda i: (i, 0)),
            pl.BlockSpec((D, D), lambda i: (0, 0)),
            pl.BlockSpec((1, D), lambda i: (0, 0)),
        ],
        out_specs=pl.BlockSpec((_BR, D), lambda i: (i, 0)),
        out_shape=jax.ShapeDtypeStruct((n_out, D), jnp.float32),
    )(agg, deg, w, b)


def kernel(x, edge_index, edge_weight, W1, b1, W2, b2):
    src = edge_index[0]
    dst = edge_index[1]
    pad = E_PAD - E
    ar = jnp.arange(pad, dtype=jnp.int32)
    src_p = jnp.concatenate([src, ar % N])
    dst_p = jnp.concatenate([dst, N + ar % (N_PAD - N)])
    w_p = jnp.concatenate([edge_weight, jnp.zeros((pad,), jnp.float32)])
    dst2d = dst_p.reshape(E_PAD // _ER, _ER)
    rec = 3 * ECB
    sd2 = jnp.concatenate(
        [src_p.reshape(NW, NCHK, ECB, B),
         dst_p.reshape(NW, NCHK, ECB, B),
         lax.bitcast_convert_type(w_p, jnp.int32).reshape(NW, NCHK, ECB, B)],
        axis=2).reshape(NW * NCHK * rec, B)
    sd2 = jnp.concatenate([sd2, jnp.zeros((rec, B), jnp.int32)])
    zrows = jnp.zeros((_CH, D), jnp.float32)

    agg1 = _make_sc_agg()(x, sd2, zrows)
    deg_col = _tc_degree(dst2d).reshape(N_PAD, 1)
    h = _tc_layer(agg1, deg_col, W1, b1.reshape(1, D), relu=True, n_out=N_PAD)
    agg2 = _make_sc_agg()(h, sd2, zrows)
    out = _tc_layer(agg2, deg_col, W2, b2.reshape(1, D), relu=False, n_out=N)
    return out
